# baseline (speedup 1.0000x reference)
import jax
import jax.numpy as jnp
from jax import lax
from jax.experimental import pallas as pl
from jax.experimental.pallas import tpu as pltpu
from jax.experimental.pallas import tpu_sc as plsc

HIDDEN = 768
EPS = 1e-12
LANES = 16
NV = HIDDEN // LANES

B_TOTAL = 4 * 8192
NUM_WORKERS = 32
ROWS_PER_WORKER = B_TOTAL // NUM_WORKERS
CHUNK = 32
NCHUNKS = ROWS_PER_WORKER // CHUNK
NBUF = 4
NR = 4

_GATHER_DNUMS = lax.GatherDimensionNumbers(
    offset_dims=(), collapsed_slice_dims=(0,), start_index_map=(0,)
)


def _lane_shuffle(v, perm):
    return lax.gather(
        v,
        perm[:, None],
        _GATHER_DNUMS,
        slice_sizes=(1,),
        mode=lax.GatherScatterMode.PROMISE_IN_BOUNDS,
    )


def _rsqrt(x):
    i = lax.bitcast_convert_type(x, jnp.int32)
    i = jnp.int32(0x5F3759DF) - lax.shift_right_logical(i, jnp.int32(1))
    y = lax.bitcast_convert_type(i, jnp.float32)
    half_x = x * jnp.float32(0.5)
    for _ in range(3):
        y = y * (jnp.float32(1.5) - half_x * y * y)
    return y


def _ln_chunk(rows_v):
    inv_h = jnp.float32(1.0 / HIDDEN)
    idx16 = lax.iota(jnp.int32, LANES)
    perms = [lax.bitwise_xor(idx16, jnp.int32(sh)) for sh in (8, 4, 2, 1)]
    zero = jnp.zeros((LANES,), jnp.float32)

    @plsc.parallel_loop(0, CHUNK, step=NR, unroll=2)
    def _(r0):
        rs = [r0 + k for k in range(NR)]
        s = [zero] * NR
        q = [zero] * NR
        for j in range(NV):
            for k in range(NR):
                v = rows_v[rs[k], pl.ds(j * LANES, LANES)]
                s[k] = s[k] + v
                q[k] = q[k] + v * v
        for pm in perms:
            for k in range(NR):
                s[k] = s[k] + _lane_shuffle(s[k], pm)
                q[k] = q[k] + _lane_shuffle(q[k], pm)
        mu = [s[k] * inv_h for k in range(NR)]
        y = [
            _rsqrt(q[k] * inv_h - mu[k] * mu[k] + jnp.float32(EPS))
            for k in range(NR)
        ]
        for j in range(NV):
            for k in range(NR):
                v = rows_v[rs[k], pl.ds(j * LANES, LANES)]
                rows_v[rs[k], pl.ds(j * LANES, LANES)] = (v - mu[k]) * y[k]


def _sc_kernel(ids_hbm, table_hbm, out_hbm, idx_v,
               rows0, rows1, rows2, rows3,
               gs0, gs1, gs2, gs3, os0, os1, os2, os3):
    nc = 2
    wid = lax.axis_index("s") * nc + lax.axis_index("c")
    base = wid * ROWS_PER_WORKER
    pltpu.sync_copy(ids_hbm.at[pl.ds(base, ROWS_PER_WORKER)], idx_v)

    bufs = (rows0, rows1, rows2, rows3)
    gsems = (gs0, gs1, gs2, gs3)
    osems = (os0, os1, os2, os3)

    for g in range(NBUF - 1):
        pltpu.async_copy(
            table_hbm.at[idx_v.at[pl.ds(g * CHUNK, CHUNK)]], bufs[g], gsems[g]
        )

    def ring_body(p, _):
        for b in range(NBUF):
            g = p * NBUF + b
            buf, gsem, osem = bufs[b], gsems[b], osems[b]

            pltpu.make_async_copy(
                out_hbm.at[pl.ds(base, CHUNK)], buf, gsem
            ).wait()
            _ln_chunk(buf)

            nb = (b + NBUF - 1) % NBUF
            @pl.when(g + NBUF - 1 < NCHUNKS)
            def _():
                @pl.when(g >= 1)
                def _():
                    pltpu.make_async_copy(
                        bufs[nb], out_hbm.at[pl.ds(base, CHUNK)], osems[nb]
                    ).wait()
                pltpu.async_copy(
                    table_hbm.at[
                        idx_v.at[pl.ds((g + NBUF - 1) * CHUNK, CHUNK)]
                    ],
                    bufs[nb], gsems[nb],
                )

            pltpu.async_copy(
                buf, out_hbm.at[pl.ds(base + g * CHUNK, CHUNK)], osem
            )
        return 0

    lax.fori_loop(0, NCHUNKS // NBUF, ring_body, 0)

    for b in range(NBUF):
        pltpu.make_async_copy(
            bufs[b], out_hbm.at[pl.ds(base, CHUNK)], osems[b]
        ).wait()


@jax.jit
def _run(ids_flat, table):
    mesh = plsc.VectorSubcoreMesh(core_axis_name="c", subcore_axis_name="s")
    f = pl.kernel(
        _sc_kernel,
        mesh=mesh,
        out_type=jax.ShapeDtypeStruct((B_TOTAL, HIDDEN), jnp.float32),
        scratch_types=[
            pltpu.VMEM((ROWS_PER_WORKER,), jnp.int32),
            pltpu.VMEM((CHUNK, HIDDEN), jnp.float32),
            pltpu.VMEM((CHUNK, HIDDEN), jnp.float32),
            pltpu.VMEM((CHUNK, HIDDEN), jnp.float32),
            pltpu.VMEM((CHUNK, HIDDEN), jnp.float32),
            pltpu.SemaphoreType.DMA,
            pltpu.SemaphoreType.DMA,
            pltpu.SemaphoreType.DMA,
            pltpu.SemaphoreType.DMA,
            pltpu.SemaphoreType.DMA,
            pltpu.SemaphoreType.DMA,
            pltpu.SemaphoreType.DMA,
            pltpu.SemaphoreType.DMA,
        ],
    )
    return f(ids_flat, table)


def kernel(input_ids, table):
    ids_flat = input_ids.reshape(-1).astype(jnp.int32)
    out = _run(ids_flat, table)
    return out.reshape(input_ids.shape + (HIDDEN,))


# speedup vs baseline: 1.0482x; 1.0482x over previous
import jax
import jax.numpy as jnp
from jax import lax
from jax.experimental import pallas as pl
from jax.experimental.pallas import tpu as pltpu
from jax.experimental.pallas import tpu_sc as plsc

HIDDEN = 768
EPS = 1e-12
LANES = 16
NV = HIDDEN // LANES

B_TOTAL = 4 * 8192
NUM_WORKERS = 32
ROWS_PER_WORKER = B_TOTAL // NUM_WORKERS
CHUNK = 32
NCHUNKS = ROWS_PER_WORKER // CHUNK
NBUF = 4
NR = 8

_GATHER_DNUMS = lax.GatherDimensionNumbers(
    offset_dims=(), collapsed_slice_dims=(0,), start_index_map=(0,)
)


def _lane_shuffle(v, perm):
    return lax.gather(
        v,
        perm[:, None],
        _GATHER_DNUMS,
        slice_sizes=(1,),
        mode=lax.GatherScatterMode.PROMISE_IN_BOUNDS,
    )


def _rsqrt(x):
    i = lax.bitcast_convert_type(x, jnp.int32)
    i = jnp.int32(0x5F3759DF) - lax.shift_right_logical(i, jnp.int32(1))
    y = lax.bitcast_convert_type(i, jnp.float32)
    half_x = x * jnp.float32(0.5)
    for _ in range(3):
        y = y * (jnp.float32(1.5) - half_x * y * y)
    return y


def _ln_chunk(rows_v):
    inv_h = jnp.float32(1.0 / HIDDEN)
    idx16 = lax.iota(jnp.int32, LANES)
    perms = [lax.bitwise_xor(idx16, jnp.int32(sh)) for sh in (8, 4, 2, 1)]
    zero = jnp.zeros((LANES,), jnp.float32)

    @plsc.parallel_loop(0, CHUNK, step=NR)
    def _(r0):
        rs = [r0 + k for k in range(NR)]
        s = [zero] * NR
        q = [zero] * NR
        for j in range(NV):
            for k in range(NR):
                v = rows_v[rs[k], pl.ds(j * LANES, LANES)]
                s[k] = s[k] + v
                q[k] = q[k] + v * v
        for pm in perms:
            for k in range(NR):
                s[k] = s[k] + _lane_shuffle(s[k], pm)
                q[k] = q[k] + _lane_shuffle(q[k], pm)
        mu = [s[k] * inv_h for k in range(NR)]
        y = [
            _rsqrt(q[k] * inv_h - mu[k] * mu[k] + jnp.float32(EPS))
            for k in range(NR)
        ]
        for j in range(NV):
            for k in range(NR):
                v = rows_v[rs[k], pl.ds(j * LANES, LANES)]
                rows_v[rs[k], pl.ds(j * LANES, LANES)] = (v - mu[k]) * y[k]


def _sc_kernel(ids_hbm, table_hbm, out_hbm, idx_v,
               rows0, rows1, rows2, rows3,
               gs0, gs1, gs2, gs3, os0, os1, os2, os3):
    nc = 2
    wid = lax.axis_index("s") * nc + lax.axis_index("c")
    base = wid * ROWS_PER_WORKER
    pltpu.sync_copy(ids_hbm.at[pl.ds(base, ROWS_PER_WORKER)], idx_v)

    bufs = (rows0, rows1, rows2, rows3)
    gsems = (gs0, gs1, gs2, gs3)
    osems = (os0, os1, os2, os3)

    for g in range(NBUF - 1):
        pltpu.async_copy(
            table_hbm.at[idx_v.at[pl.ds(g * CHUNK, CHUNK)]], bufs[g], gsems[g]
        )

    def ring_body(p, _):
        for b in range(NBUF):
            g = p * NBUF + b
            buf, gsem, osem = bufs[b], gsems[b], osems[b]

            pltpu.make_async_copy(
                out_hbm.at[pl.ds(base, CHUNK)], buf, gsem
            ).wait()
            _ln_chunk(buf)

            nb = (b + NBUF - 1) % NBUF
            @pl.when(g + NBUF - 1 < NCHUNKS)
            def _():
                @pl.when(g >= 1)
                def _():
                    pltpu.make_async_copy(
                        bufs[nb], out_hbm.at[pl.ds(base, CHUNK)], osems[nb]
                    ).wait()
                pltpu.async_copy(
                    table_hbm.at[
                        idx_v.at[pl.ds((g + NBUF - 1) * CHUNK, CHUNK)]
                    ],
                    bufs[nb], gsems[nb],
                )

            pltpu.async_copy(
                buf, out_hbm.at[pl.ds(base + g * CHUNK, CHUNK)], osem
            )
        return 0

    lax.fori_loop(0, NCHUNKS // NBUF, ring_body, 0)

    for b in range(NBUF):
        pltpu.make_async_copy(
            bufs[b], out_hbm.at[pl.ds(base, CHUNK)], osems[b]
        ).wait()


@jax.jit
def _run(ids_flat, table):
    mesh = plsc.VectorSubcoreMesh(core_axis_name="c", subcore_axis_name="s")
    f = pl.kernel(
        _sc_kernel,
        mesh=mesh,
        out_type=jax.ShapeDtypeStruct((B_TOTAL, HIDDEN), jnp.float32),
        scratch_types=[
            pltpu.VMEM((ROWS_PER_WORKER,), jnp.int32),
            pltpu.VMEM((CHUNK, HIDDEN), jnp.float32),
            pltpu.VMEM((CHUNK, HIDDEN), jnp.float32),
            pltpu.VMEM((CHUNK, HIDDEN), jnp.float32),
            pltpu.VMEM((CHUNK, HIDDEN), jnp.float32),
            pltpu.SemaphoreType.DMA,
            pltpu.SemaphoreType.DMA,
            pltpu.SemaphoreType.DMA,
            pltpu.SemaphoreType.DMA,
            pltpu.SemaphoreType.DMA,
            pltpu.SemaphoreType.DMA,
            pltpu.SemaphoreType.DMA,
            pltpu.SemaphoreType.DMA,
        ],
    )
    return f(ids_flat, table)


def kernel(input_ids, table):
    ids_flat = input_ids.reshape(-1).astype(jnp.int32)
    out = _run(ids_flat, table)
    return out.reshape(input_ids.shape + (HIDDEN,))


# speedup vs baseline: 1.1703x; 1.1165x over previous
import jax
import jax.numpy as jnp
from jax import lax
from jax.experimental import pallas as pl
from jax.experimental.pallas import tpu as pltpu
from jax.experimental.pallas import tpu_sc as plsc

HIDDEN = 768
EPS = 1e-12
LANES = 16
NV = HIDDEN // LANES

B_TOTAL = 4 * 8192
NUM_WORKERS = 32
ROWS_PER_WORKER = B_TOTAL // NUM_WORKERS
CHUNK = 32
NCHUNKS = ROWS_PER_WORKER // CHUNK
NBUF = 4
NR = 2

_GATHER_DNUMS = lax.GatherDimensionNumbers(
    offset_dims=(), collapsed_slice_dims=(0,), start_index_map=(0,)
)


def _lane_shuffle(v, perm):
    return lax.gather(
        v,
        perm[:, None],
        _GATHER_DNUMS,
        slice_sizes=(1,),
        mode=lax.GatherScatterMode.PROMISE_IN_BOUNDS,
    )


def _rsqrt(x):
    i = lax.bitcast_convert_type(x, jnp.int32)
    i = jnp.int32(0x5F3759DF) - lax.shift_right_logical(i, jnp.int32(1))
    y = lax.bitcast_convert_type(i, jnp.float32)
    half_x = x * jnp.float32(0.5)
    for _ in range(3):
        y = y * (jnp.float32(1.5) - half_x * y * y)
    return y


def _ln_chunk(rows_v):
    inv_h = jnp.float32(1.0 / HIDDEN)
    idx16 = lax.iota(jnp.int32, LANES)
    perms = [lax.bitwise_xor(idx16, jnp.int32(sh)) for sh in (8, 4, 2, 1)]
    zero = jnp.zeros((LANES,), jnp.float32)

    @plsc.parallel_loop(0, CHUNK, step=NR)
    def _(r0):
        rs = [r0 + k for k in range(NR)]
        s = [zero] * NR
        q = [zero] * NR
        for j in range(NV):
            for k in range(NR):
                v = rows_v[rs[k], pl.ds(j * LANES, LANES)]
                s[k] = s[k] + v
                q[k] = q[k] + v * v
        for pm in perms:
            for k in range(NR):
                s[k] = s[k] + _lane_shuffle(s[k], pm)
                q[k] = q[k] + _lane_shuffle(q[k], pm)
        mu = [s[k] * inv_h for k in range(NR)]
        y = [
            _rsqrt(q[k] * inv_h - mu[k] * mu[k] + jnp.float32(EPS))
            for k in range(NR)
        ]
        for j in range(NV):
            for k in range(NR):
                v = rows_v[rs[k], pl.ds(j * LANES, LANES)]
                rows_v[rs[k], pl.ds(j * LANES, LANES)] = (v - mu[k]) * y[k]


def _sc_kernel(ids_hbm, table_hbm, out_hbm, idx_v,
               rows0, rows1, rows2, rows3,
               gs0, gs1, gs2, gs3, os0, os1, os2, os3):
    nc = 2
    wid = lax.axis_index("s") * nc + lax.axis_index("c")
    base = wid * ROWS_PER_WORKER
    pltpu.sync_copy(ids_hbm.at[pl.ds(base, ROWS_PER_WORKER)], idx_v)

    bufs = (rows0, rows1, rows2, rows3)
    gsems = (gs0, gs1, gs2, gs3)
    osems = (os0, os1, os2, os3)

    for g in range(NBUF - 1):
        pltpu.async_copy(
            table_hbm.at[idx_v.at[pl.ds(g * CHUNK, CHUNK)]], bufs[g], gsems[g]
        )

    def ring_body(p, _):
        for b in range(NBUF):
            g = p * NBUF + b
            buf, gsem, osem = bufs[b], gsems[b], osems[b]

            pltpu.make_async_copy(
                out_hbm.at[pl.ds(base, CHUNK)], buf, gsem
            ).wait()
            _ln_chunk(buf)

            nb = (b + NBUF - 1) % NBUF
            @pl.when(g + NBUF - 1 < NCHUNKS)
            def _():
                @pl.when(g >= 1)
                def _():
                    pltpu.make_async_copy(
                        bufs[nb], out_hbm.at[pl.ds(base, CHUNK)], osems[nb]
                    ).wait()
                pltpu.async_copy(
                    table_hbm.at[
                        idx_v.at[pl.ds((g + NBUF - 1) * CHUNK, CHUNK)]
                    ],
                    bufs[nb], gsems[nb],
                )

            pltpu.async_copy(
                buf, out_hbm.at[pl.ds(base + g * CHUNK, CHUNK)], osem
            )
        return 0

    lax.fori_loop(0, NCHUNKS // NBUF, ring_body, 0)

    for b in range(NBUF):
        pltpu.make_async_copy(
            bufs[b], out_hbm.at[pl.ds(base, CHUNK)], osems[b]
        ).wait()


@jax.jit
def _run(ids_flat, table):
    mesh = plsc.VectorSubcoreMesh(core_axis_name="c", subcore_axis_name="s")
    f = pl.kernel(
        _sc_kernel,
        mesh=mesh,
        out_type=jax.ShapeDtypeStruct((B_TOTAL, HIDDEN), jnp.float32),
        scratch_types=[
            pltpu.VMEM((ROWS_PER_WORKER,), jnp.int32),
            pltpu.VMEM((CHUNK, HIDDEN), jnp.float32),
            pltpu.VMEM((CHUNK, HIDDEN), jnp.float32),
            pltpu.VMEM((CHUNK, HIDDEN), jnp.float32),
            pltpu.VMEM((CHUNK, HIDDEN), jnp.float32),
            pltpu.SemaphoreType.DMA,
            pltpu.SemaphoreType.DMA,
            pltpu.SemaphoreType.DMA,
            pltpu.SemaphoreType.DMA,
            pltpu.SemaphoreType.DMA,
            pltpu.SemaphoreType.DMA,
            pltpu.SemaphoreType.DMA,
            pltpu.SemaphoreType.DMA,
        ],
    )
    return f(ids_flat, table)


def kernel(input_ids, table):
    ids_flat = input_ids.reshape(-1).astype(jnp.int32)
    out = _run(ids_flat, table)
    return out.reshape(input_ids.shape + (HIDDEN,))


# speedup vs baseline: 1.2402x; 1.0598x over previous
import jax
import jax.numpy as jnp
from jax import lax
from jax.experimental import pallas as pl
from jax.experimental.pallas import tpu as pltpu
from jax.experimental.pallas import tpu_sc as plsc

HIDDEN = 768
EPS = 1e-12
LANES = 16
NV = HIDDEN // LANES

B_TOTAL = 4 * 8192
NUM_WORKERS = 32
ROWS_PER_WORKER = B_TOTAL // NUM_WORKERS
CHUNK = 32
NCHUNKS = ROWS_PER_WORKER // CHUNK
NBUF = 4
NR = 4

_GATHER_DNUMS = lax.GatherDimensionNumbers(
    offset_dims=(), collapsed_slice_dims=(0,), start_index_map=(0,)
)


def _lane_shuffle(v, perm):
    return lax.gather(
        v,
        perm[:, None],
        _GATHER_DNUMS,
        slice_sizes=(1,),
        mode=lax.GatherScatterMode.PROMISE_IN_BOUNDS,
    )


def _rsqrt(x):
    i = lax.bitcast_convert_type(x, jnp.int32)
    i = jnp.int32(0x5F3759DF) - lax.shift_right_logical(i, jnp.int32(1))
    y = lax.bitcast_convert_type(i, jnp.float32)
    half_x = x * jnp.float32(0.5)
    for _ in range(3):
        y = y * (jnp.float32(1.5) - half_x * y * y)
    return y


def _ln_chunk(rows_v):
    inv_h = jnp.float32(1.0 / HIDDEN)
    idx16 = lax.iota(jnp.int32, LANES)
    perms = [lax.bitwise_xor(idx16, jnp.int32(sh)) for sh in (8, 4, 2, 1)]
    zero = jnp.zeros((LANES,), jnp.float32)

    @plsc.parallel_loop(0, CHUNK, step=NR)
    def _(r0):
        rs = [r0 + k for k in range(NR)]
        s = [zero] * NR
        q = [zero] * NR
        for j in range(NV):
            for k in range(NR):
                v = rows_v[rs[k], pl.ds(j * LANES, LANES)]
                s[k] = s[k] + v
                q[k] = q[k] + v * v
        for pm in perms:
            for k in range(NR):
                s[k] = s[k] + _lane_shuffle(s[k], pm)
                q[k] = q[k] + _lane_shuffle(q[k], pm)
        mu = [s[k] * inv_h for k in range(NR)]
        y = [
            _rsqrt(q[k] * inv_h - mu[k] * mu[k] + jnp.float32(EPS))
            for k in range(NR)
        ]
        for j in range(NV):
            for k in range(NR):
                v = rows_v[rs[k], pl.ds(j * LANES, LANES)]
                rows_v[rs[k], pl.ds(j * LANES, LANES)] = (v - mu[k]) * y[k]


def _sc_kernel(ids_hbm, table_hbm, out_hbm, idx_v,
               rows0, rows1, rows2, rows3,
               gs0, gs1, gs2, gs3, os0, os1, os2, os3):
    nc = 2
    wid = lax.axis_index("s") * nc + lax.axis_index("c")
    base = wid * ROWS_PER_WORKER
    pltpu.sync_copy(ids_hbm.at[pl.ds(base, ROWS_PER_WORKER)], idx_v)

    bufs = (rows0, rows1, rows2, rows3)
    gsems = (gs0, gs1, gs2, gs3)
    osems = (os0, os1, os2, os3)

    for g in range(NBUF - 1):
        pltpu.async_copy(
            table_hbm.at[idx_v.at[pl.ds(g * CHUNK, CHUNK)]], bufs[g], gsems[g]
        )

    def ring_body(p, _):
        for b in range(NBUF):
            g = p * NBUF + b
            buf, gsem, osem = bufs[b], gsems[b], osems[b]

            pltpu.make_async_copy(
                out_hbm.at[pl.ds(base, CHUNK)], buf, gsem
            ).wait()
            _ln_chunk(buf)

            nb = (b + NBUF - 1) % NBUF
            @pl.when(g + NBUF - 1 < NCHUNKS)
            def _():
                @pl.when(g >= 1)
                def _():
                    pltpu.make_async_copy(
                        bufs[nb], out_hbm.at[pl.ds(base, CHUNK)], osems[nb]
                    ).wait()
                pltpu.async_copy(
                    table_hbm.at[
                        idx_v.at[pl.ds((g + NBUF - 1) * CHUNK, CHUNK)]
                    ],
                    bufs[nb], gsems[nb],
                )

            pltpu.async_copy(
                buf, out_hbm.at[pl.ds(base + g * CHUNK, CHUNK)], osem
            )
        return 0

    lax.fori_loop(0, NCHUNKS // NBUF, ring_body, 0)

    for b in range(NBUF):
        pltpu.make_async_copy(
            bufs[b], out_hbm.at[pl.ds(base, CHUNK)], osems[b]
        ).wait()


@jax.jit
def _run(ids_flat, table):
    mesh = plsc.VectorSubcoreMesh(core_axis_name="c", subcore_axis_name="s")
    f = pl.kernel(
        _sc_kernel,
        mesh=mesh,
        out_type=jax.ShapeDtypeStruct((B_TOTAL, HIDDEN), jnp.float32),
        scratch_types=[
            pltpu.VMEM((ROWS_PER_WORKER,), jnp.int32),
            pltpu.VMEM((CHUNK, HIDDEN), jnp.float32),
            pltpu.VMEM((CHUNK, HIDDEN), jnp.float32),
            pltpu.VMEM((CHUNK, HIDDEN), jnp.float32),
            pltpu.VMEM((CHUNK, HIDDEN), jnp.float32),
            pltpu.SemaphoreType.DMA,
            pltpu.SemaphoreType.DMA,
            pltpu.SemaphoreType.DMA,
            pltpu.SemaphoreType.DMA,
            pltpu.SemaphoreType.DMA,
            pltpu.SemaphoreType.DMA,
            pltpu.SemaphoreType.DMA,
            pltpu.SemaphoreType.DMA,
        ],
    )
    return f(ids_flat, table)


def kernel(input_ids, table):
    ids_flat = input_ids.reshape(-1).astype(jnp.int32)
    out = _run(ids_flat, table)
    return out.reshape(input_ids.shape + (HIDDEN,))
